# Initial kernel scaffold; baseline (speedup 1.0000x reference)
import jax
import jax.numpy as jnp
from jax.experimental import pallas as pl


def kernel(src, tgt, src_coords, tgt_coords, src_shortcut, tgt_shortcut, src_shortcut_coords, tgt_shortcut_coords):
    raise NotImplementedError("write your pallas kernel here")


# speedup vs baseline: 1.6641x; 1.6641x over previous
import functools

import jax
import jax.numpy as jnp
from jax import lax
from jax.experimental import pallas as pl
from jax.experimental.pallas import tpu as pltpu
from jax.experimental.pallas import tpu_sc as plsc

N = 8192
NF = 16384
C = 256
BQ = 128
NBLK = NF // BQ

GCHUNK = 128


def _nn_body(q_ref, ct_ref, out_ref):
    d0 = q_ref[0, :, 0:1] - ct_ref[0, 0:1, :]
    d1 = q_ref[0, :, 1:2] - ct_ref[0, 1:2, :]
    d2 = q_ref[0, :, 2:3] - ct_ref[0, 2:3, :]
    dist = (d0 * d0 + d1 * d1) + d2 * d2
    m = jnp.min(dist, axis=1, keepdims=True)
    cand = lax.broadcasted_iota(jnp.int32, (BQ, N), 1)
    idx = jnp.min(jnp.where(dist == m, cand, N), axis=1)
    out_ref[0, 0, 0, :] = idx


def _nn_indices(qs, cts):
    return pl.pallas_call(
        _nn_body,
        grid=(2, NBLK),
        in_specs=[
            pl.BlockSpec((1, BQ, 3), lambda s, i: (s, i, 0)),
            pl.BlockSpec((1, 3, N), lambda s, i: (s, 0, 0)),
        ],
        out_specs=pl.BlockSpec((1, 1, 1, BQ), lambda s, i: (s, i, 0, 0)),
        out_shape=jax.ShapeDtypeStruct((2, NBLK, 1, BQ), jnp.int32),
    )(qs, cts)


def _sc_gather_concat(src, tgt, idx3):
    info = plsc.get_sparse_core_info()
    nw = info.num_cores * info.num_subcores
    copy_pw = N // nw
    gath_pw = NF // nw
    nchunk = gath_pw // GCHUNK

    mesh = plsc.VectorSubcoreMesh(core_axis_name="c", subcore_axis_name="s")

    @functools.partial(
        pl.kernel,
        mesh=mesh,
        out_type=[
            jax.ShapeDtypeStruct((N + NF, C), jnp.float32),
            jax.ShapeDtypeStruct((N + NF, C), jnp.float32),
        ],
        scratch_types=[
            pltpu.VMEM((nchunk, GCHUNK), jnp.int32),
            pltpu.VMEM((GCHUNK, C), jnp.float32),
            pltpu.VMEM((GCHUNK, C), jnp.float32),
            pltpu.SemaphoreType.DMA,
            pltpu.SemaphoreType.DMA,
        ],
    )
    def sc_k(src_hbm, tgt_hbm, idx_hbm, src_out, tgt_out,
             idx_v, buf0, buf1, sem0, sem1):
        wid = lax.axis_index("s") * info.num_cores + lax.axis_index("c")
        bufs = (buf0, buf1)
        sems = (sem0, sem1)
        for side, (tab, out) in enumerate(((src_hbm, src_out), (tgt_hbm, tgt_out))):
            cb = wid * copy_pw
            pltpu.sync_copy(tab.at[pl.ds(cb, copy_pw)], out.at[pl.ds(cb, copy_pw)])
            gb = wid * nchunk
            pltpu.sync_copy(idx_hbm.at[side, pl.ds(gb, nchunk)], idx_v)
            cps = [None] * nchunk
            cps[0] = pltpu.async_copy(tab.at[idx_v.at[0]], bufs[0], sems[0])
            for c in range(nchunk):
                if c + 1 < nchunk:
                    cps[c + 1] = pltpu.async_copy(
                        tab.at[idx_v.at[c + 1]], bufs[(c + 1) % 2], sems[(c + 1) % 2])
                cps[c].wait()
                pltpu.sync_copy(
                    bufs[c % 2],
                    out.at[pl.ds(N + (gb + c) * GCHUNK, GCHUNK)])

    return sc_k(src, tgt, idx3)


def kernel(src, tgt, src_coords, tgt_coords, src_shortcut, tgt_shortcut,
           src_shortcut_coords, tgt_shortcut_coords):
    qs = jnp.stack([src_shortcut_coords, tgt_shortcut_coords])
    cts = jnp.stack([src_coords.T, tgt_coords.T])
    idx = _nn_indices(qs, cts)
    idx3 = idx.reshape(2, NF // GCHUNK, GCHUNK)
    src_out, tgt_out = _sc_gather_concat(src, tgt, idx3)
    return (src_out, tgt_out, src_shortcut_coords, tgt_shortcut_coords)


# speedup vs baseline: 1.6781x; 1.0085x over previous
import functools

import jax
import jax.numpy as jnp
from jax import lax
from jax.experimental import pallas as pl
from jax.experimental.pallas import tpu as pltpu
from jax.experimental.pallas import tpu_sc as plsc

N = 8192
NF = 16384
C = 256
BQ = 128
NBLK = NF // BQ

GCHUNK = 128


def _nn_body(q_ref, ct_ref, out_ref):
    d0 = q_ref[0, :, 0:1] - ct_ref[0, 0:1, :]
    d1 = q_ref[0, :, 1:2] - ct_ref[0, 1:2, :]
    d2 = q_ref[0, :, 2:3] - ct_ref[0, 2:3, :]
    dist = (d0 * d0 + d1 * d1) + d2 * d2
    m = jnp.min(dist, axis=1, keepdims=True)
    cand = lax.broadcasted_iota(jnp.int32, (BQ, N), 1)
    idx = jnp.min(jnp.where(dist == m, cand, N), axis=1)
    out_ref[0, 0, 0, :] = idx


def _nn_indices(qs, cts):
    return pl.pallas_call(
        _nn_body,
        grid=(2, NBLK),
        in_specs=[
            pl.BlockSpec((1, BQ, 3), lambda s, i: (s, i, 0)),
            pl.BlockSpec((1, 3, N), lambda s, i: (s, 0, 0)),
        ],
        out_specs=pl.BlockSpec((1, 1, 1, BQ), lambda s, i: (s, i, 0, 0)),
        out_shape=jax.ShapeDtypeStruct((2, NBLK, 1, BQ), jnp.int32),
    )(qs, cts)


def _sc_gather_concat(src, tgt, idx3):
    info = plsc.get_sparse_core_info()
    nw = info.num_cores * info.num_subcores
    copy_pw = N // nw
    gath_pw = NF // nw
    nchunk = gath_pw // GCHUNK

    njobs = 2 * nchunk
    NBUF = 3

    mesh = plsc.VectorSubcoreMesh(core_axis_name="c", subcore_axis_name="s")

    @functools.partial(
        pl.kernel,
        mesh=mesh,
        out_type=[
            jax.ShapeDtypeStruct((N + NF, C), jnp.float32),
            jax.ShapeDtypeStruct((N + NF, C), jnp.float32),
        ],
        scratch_types=[
            pltpu.VMEM((2, nchunk, GCHUNK), jnp.int32),
        ]
        + [pltpu.VMEM((GCHUNK, C), jnp.float32)] * NBUF
        + [pltpu.SemaphoreType.DMA] * (2 * NBUF + 3),
    )
    def sc_k(src_hbm, tgt_hbm, idx_hbm, src_out, tgt_out, idx_v, *rest):
        bufs = rest[:NBUF]
        gsem = rest[NBUF:2 * NBUF]
        ssem = rest[2 * NBUF:3 * NBUF]
        hsem = rest[3 * NBUF:3 * NBUF + 2]
        isem = rest[3 * NBUF + 2]
        wid = lax.axis_index("s") * info.num_cores + lax.axis_index("c")
        tabs = (src_hbm, tgt_hbm)
        outs = (src_out, tgt_out)
        gb = wid * nchunk

        cb = wid * copy_pw
        head = [
            pltpu.async_copy(tabs[s].at[pl.ds(cb, copy_pw)],
                             outs[s].at[pl.ds(cb, copy_pw)], hsem[s])
            for s in range(2)
        ]
        pltpu.async_copy(idx_hbm.at[0, pl.ds(gb, nchunk)], idx_v.at[0], isem).wait()
        pltpu.async_copy(idx_hbm.at[1, pl.ds(gb, nchunk)], idx_v.at[1], isem).wait()

        gcp = [None] * njobs
        scp = [None] * njobs

        def fire(g):
            s, c = divmod(g, nchunk)
            gcp[g] = pltpu.async_copy(
                tabs[s].at[idx_v.at[s, c]], bufs[g % NBUF], gsem[g % NBUF])

        for g in range(min(NBUF, njobs)):
            fire(g)
        for g in range(njobs):
            s, c = divmod(g, nchunk)
            gcp[g].wait()
            scp[g] = pltpu.async_copy(
                bufs[g % NBUF],
                outs[s].at[pl.ds(N + (gb + c) * GCHUNK, GCHUNK)],
                ssem[g % NBUF])
            if g + NBUF < njobs:
                scp[g].wait()
                fire(g + NBUF)
        for g in range(max(0, njobs - NBUF), njobs):
            scp[g].wait()
        for s in range(2):
            head[s].wait()

    return sc_k(src, tgt, idx3)


def kernel(src, tgt, src_coords, tgt_coords, src_shortcut, tgt_shortcut,
           src_shortcut_coords, tgt_shortcut_coords):
    qs = jnp.stack([src_shortcut_coords, tgt_shortcut_coords])
    cts = jnp.stack([src_coords.T, tgt_coords.T])
    idx = _nn_indices(qs, cts)
    idx3 = idx.reshape(2, NF // GCHUNK, GCHUNK)
    src_out, tgt_out = _sc_gather_concat(src, tgt, idx3)
    return (src_out, tgt_out, src_shortcut_coords, tgt_shortcut_coords)


# speedup vs baseline: 2.8479x; 1.6971x over previous
import functools

import jax
import jax.numpy as jnp
from jax import lax
from jax.experimental import pallas as pl
from jax.experimental.pallas import tpu as pltpu
from jax.experimental.pallas import tpu_sc as plsc

N = 8192
NF = 16384
C = 256
BQ = 128
NBLK = NF // BQ

GCHUNK = 128


def _nn_body(q_ref, ct_ref, out_ref):
    d0 = q_ref[0, :, 0:1] - ct_ref[0, 0:1, :]
    d1 = q_ref[0, :, 1:2] - ct_ref[0, 1:2, :]
    d2 = q_ref[0, :, 2:3] - ct_ref[0, 2:3, :]
    dist = (d0 * d0 + d1 * d1) + d2 * d2
    m = jnp.min(dist, axis=1, keepdims=True)
    cand = lax.broadcasted_iota(jnp.int32, (BQ, N), 1)
    idx = jnp.min(jnp.where(dist == m, cand, N), axis=1)
    out_ref[0, 0, 0, :] = idx


def _nn_indices(qs, cts):
    return pl.pallas_call(
        _nn_body,
        grid=(2, NBLK),
        in_specs=[
            pl.BlockSpec((1, BQ, 3), lambda s, i: (s, i, 0)),
            pl.BlockSpec((1, 3, N), lambda s, i: (s, 0, 0)),
        ],
        out_specs=pl.BlockSpec((1, 1, 1, BQ), lambda s, i: (s, i, 0, 0)),
        out_shape=jax.ShapeDtypeStruct((2, NBLK, 1, BQ), jnp.int32),
    )(qs, cts)


def _sc_gather_concat(src, tgt, idx3):
    info = plsc.get_sparse_core_info()
    nw = info.num_cores * info.num_subcores
    copy_pw = N // nw
    gath_pw = NF // nw
    nchunk = gath_pw // GCHUNK

    cchunk = copy_pw // GCHUNK
    jside = nchunk + cchunk
    njobs = 2 * jside
    NBUF = 3

    mesh = plsc.VectorSubcoreMesh(core_axis_name="c", subcore_axis_name="s")

    @functools.partial(
        pl.kernel,
        mesh=mesh,
        out_type=[
            jax.ShapeDtypeStruct((N + NF, C), jnp.float32),
            jax.ShapeDtypeStruct((N + NF, C), jnp.float32),
        ],
        scratch_types=[
            pltpu.VMEM((2, nchunk, GCHUNK), jnp.int32),
        ]
        + [pltpu.VMEM((GCHUNK, C), jnp.float32)] * NBUF
        + [pltpu.SemaphoreType.DMA] * (2 * NBUF + 1),
    )
    def sc_k(src_hbm, tgt_hbm, idx_hbm, src_out, tgt_out, idx_v, *rest):
        bufs = rest[:NBUF]
        gsem = rest[NBUF:2 * NBUF]
        ssem = rest[2 * NBUF:3 * NBUF]
        isem = rest[3 * NBUF]
        wid = lax.axis_index("s") * info.num_cores + lax.axis_index("c")
        tabs = (src_hbm, tgt_hbm)
        outs = (src_out, tgt_out)
        gb = wid * nchunk
        cb = wid * copy_pw

        pltpu.async_copy(idx_hbm.at[0, pl.ds(gb, nchunk)], idx_v.at[0], isem).wait()
        pltpu.async_copy(idx_hbm.at[1, pl.ds(gb, nchunk)], idx_v.at[1], isem).wait()

        gcp = [None] * njobs
        scp = [None] * njobs

        def job(g):
            s, k = divmod(g, jside)
            if k < nchunk:
                return tabs[s].at[idx_v.at[s, k]], outs[s], N + (gb + k) * GCHUNK
            r = cb + (k - nchunk) * GCHUNK
            return tabs[s].at[pl.ds(r, GCHUNK)], outs[s], r

        def fire(g):
            src_ref, _, _ = job(g)
            gcp[g] = pltpu.async_copy(src_ref, bufs[g % NBUF], gsem[g % NBUF])

        for g in range(min(NBUF, njobs)):
            fire(g)
        for g in range(njobs):
            _, out, row = job(g)
            gcp[g].wait()
            scp[g] = pltpu.async_copy(
                bufs[g % NBUF], out.at[pl.ds(row, GCHUNK)], ssem[g % NBUF])
            if g + NBUF < njobs:
                scp[g].wait()
                fire(g + NBUF)
        for g in range(max(0, njobs - NBUF), njobs):
            scp[g].wait()

    return sc_k(src, tgt, idx3)


def kernel(src, tgt, src_coords, tgt_coords, src_shortcut, tgt_shortcut,
           src_shortcut_coords, tgt_shortcut_coords):
    qs = jnp.stack([src_shortcut_coords, tgt_shortcut_coords])
    cts = jnp.stack([src_coords.T, tgt_coords.T])
    idx = _nn_indices(qs, cts)
    idx3 = idx.reshape(2, NF // GCHUNK, GCHUNK)
    src_out, tgt_out = _sc_gather_concat(src, tgt, idx3)
    return (src_out, tgt_out, src_shortcut_coords, tgt_shortcut_coords)


# speedup vs baseline: 3.8405x; 1.3485x over previous
import functools

import jax
import jax.numpy as jnp
from jax import lax
from jax.experimental import pallas as pl
from jax.experimental.pallas import tpu as pltpu
from jax.experimental.pallas import tpu_sc as plsc

N = 8192
NF = 16384
C = 256
BQ = 512
NBLK = NF // BQ

GCHUNK = 128


CB = 128


def _nn_body(q_ref, ct_ref, out_ref):
    q0 = q_ref[0, :, 0:1]
    q1 = q_ref[0, :, 1:2]
    q2 = q_ref[0, :, 2:3]
    lane = lax.broadcasted_iota(jnp.int32, (BQ, CB), 1)
    runmin = jnp.full((BQ, CB), jnp.inf, jnp.float32)
    runidx = jnp.zeros((BQ, CB), jnp.int32)
    for c in range(N // CB):
        d0 = q0 - ct_ref[0, 0:1, pl.ds(c * CB, CB)]
        d1 = q1 - ct_ref[0, 1:2, pl.ds(c * CB, CB)]
        d2 = q2 - ct_ref[0, 2:3, pl.ds(c * CB, CB)]
        dc = (d0 * d0 + d1 * d1) + d2 * d2
        better = dc < runmin
        runmin = jnp.minimum(runmin, dc)
        runidx = jnp.where(better, lane + c * CB, runidx)
    m = jnp.min(runmin, axis=1, keepdims=True)
    idx = jnp.min(jnp.where(runmin == m, runidx, N), axis=1)
    out_ref[0, 0, 0, :] = idx


def _nn_indices(qs, cts):
    return pl.pallas_call(
        _nn_body,
        grid=(2, NBLK),
        in_specs=[
            pl.BlockSpec((1, BQ, 3), lambda s, i: (s, i, 0)),
            pl.BlockSpec((1, 3, N), lambda s, i: (s, 0, 0)),
        ],
        out_specs=pl.BlockSpec((1, 1, 1, BQ), lambda s, i: (s, i, 0, 0)),
        out_shape=jax.ShapeDtypeStruct((2, NBLK, 1, BQ), jnp.int32),
    )(qs, cts)


def _sc_gather_concat(src, tgt, idx3):
    info = plsc.get_sparse_core_info()
    nw = info.num_cores * info.num_subcores
    copy_pw = N // nw
    gath_pw = NF // nw
    nchunk = gath_pw // GCHUNK

    cchunk = copy_pw // GCHUNK
    jside = nchunk + cchunk
    njobs = 2 * jside
    NBUF = 3

    mesh = plsc.VectorSubcoreMesh(core_axis_name="c", subcore_axis_name="s")

    @functools.partial(
        pl.kernel,
        mesh=mesh,
        out_type=[
            jax.ShapeDtypeStruct((N + NF, C), jnp.float32),
            jax.ShapeDtypeStruct((N + NF, C), jnp.float32),
        ],
        scratch_types=[
            pltpu.VMEM((2, nchunk, GCHUNK), jnp.int32),
        ]
        + [pltpu.VMEM((GCHUNK, C), jnp.float32)] * NBUF
        + [pltpu.SemaphoreType.DMA] * (2 * NBUF + 1),
    )
    def sc_k(src_hbm, tgt_hbm, idx_hbm, src_out, tgt_out, idx_v, *rest):
        bufs = rest[:NBUF]
        gsem = rest[NBUF:2 * NBUF]
        ssem = rest[2 * NBUF:3 * NBUF]
        isem = rest[3 * NBUF]
        wid = lax.axis_index("s") * info.num_cores + lax.axis_index("c")
        tabs = (src_hbm, tgt_hbm)
        outs = (src_out, tgt_out)
        gb = wid * nchunk
        cb = wid * copy_pw

        pltpu.async_copy(idx_hbm.at[0, pl.ds(gb, nchunk)], idx_v.at[0], isem).wait()
        pltpu.async_copy(idx_hbm.at[1, pl.ds(gb, nchunk)], idx_v.at[1], isem).wait()

        gcp = [None] * njobs
        scp = [None] * njobs

        def job(g):
            s, k = divmod(g, jside)
            if k < nchunk:
                return tabs[s].at[idx_v.at[s, k]], outs[s], N + (gb + k) * GCHUNK
            r = cb + (k - nchunk) * GCHUNK
            return tabs[s].at[pl.ds(r, GCHUNK)], outs[s], r

        def fire(g):
            src_ref, _, _ = job(g)
            gcp[g] = pltpu.async_copy(src_ref, bufs[g % NBUF], gsem[g % NBUF])

        for g in range(min(NBUF, njobs)):
            fire(g)
        for g in range(njobs):
            _, out, row = job(g)
            gcp[g].wait()
            scp[g] = pltpu.async_copy(
                bufs[g % NBUF], out.at[pl.ds(row, GCHUNK)], ssem[g % NBUF])
            if g + NBUF < njobs:
                scp[g].wait()
                fire(g + NBUF)
        for g in range(max(0, njobs - NBUF), njobs):
            scp[g].wait()

    return sc_k(src, tgt, idx3)


def kernel(src, tgt, src_coords, tgt_coords, src_shortcut, tgt_shortcut,
           src_shortcut_coords, tgt_shortcut_coords):
    qs = jnp.stack([src_shortcut_coords, tgt_shortcut_coords])
    cts = jnp.stack([src_coords.T, tgt_coords.T])
    idx = _nn_indices(qs, cts)
    idx3 = idx.reshape(2, NF // GCHUNK, GCHUNK)
    src_out, tgt_out = _sc_gather_concat(src, tgt, idx3)
    return (src_out, tgt_out, src_shortcut_coords, tgt_shortcut_coords)
